# baseline (speedup 1.0000x reference)
import functools

import jax
import jax.numpy as jnp
from jax import lax
from jax.experimental import pallas as pl
from jax.experimental.pallas import tpu as pltpu
from jax.experimental.pallas import tpu_sc as plsc

EMB = 32
BATCH = 16384
L = 16


@functools.cache
def _build_kernel(num_users, num_books):
    mesh = plsc.VectorSubcoreMesh(core_axis_name="c", subcore_axis_name="s")
    NC, NS = 2, 16
    NW = NC * NS
    BW = BATCH // NW
    G = BW // L

    @functools.partial(
        pl.kernel,
        mesh=mesh,
        compiler_params=pltpu.CompilerParams(
            needs_layout_passes=False, use_tc_tiling_on_sc=False),
        out_type=jax.ShapeDtypeStruct((BATCH,), jnp.float32),
        scratch_types=[
            pltpu.VMEM((BW,), jnp.int32),
            pltpu.VMEM((BW,), jnp.int32),
            pltpu.VMEM((BW, EMB), jnp.float32),
            pltpu.VMEM((BW, EMB), jnp.float32),
            pltpu.VMEM((BW,), jnp.float32),
            pltpu.VMEM((BW,), jnp.float32),
            pltpu.VMEM((BW,), jnp.float32),
            pltpu.SemaphoreType.DMA,
        ],
    )
    def k(uidx_hbm, bidx_hbm, uemb_hbm, ubias_hbm, bemb_hbm, bbias_hbm,
          out_hbm, uidx_v, bidx_v, urows_v, brows_v, ubias_v, bbias_v,
          res_v, sem):
        wid = lax.axis_index("s") * NC + lax.axis_index("c")
        base = wid * BW

        pltpu.sync_copy(uidx_hbm.at[pl.ds(base, BW)], uidx_v)
        pltpu.sync_copy(bidx_hbm.at[pl.ds(base, BW)], bidx_v)

        c0 = pltpu.async_copy(uemb_hbm.at[uidx_v], urows_v, sem)
        c1 = pltpu.async_copy(bemb_hbm.at[bidx_v], brows_v, sem)
        c2 = pltpu.async_copy(ubias_hbm.at[uidx_v], ubias_v, sem)
        c3 = pltpu.async_copy(bbias_hbm.at[bidx_v], bbias_v, sem)
        c0.wait()
        c1.wait()
        c2.wait()
        c3.wait()

        lane = lax.iota(jnp.int32, L)

        def group(g, carry):
            acc = ubias_v[pl.ds(g * L, L)] + bbias_v[pl.ds(g * L, L)]
            for r in range(L):
                row = g * L + r
                t = (urows_v[row, pl.ds(0, L)] * brows_v[row, pl.ds(0, L)]
                     + urows_v[row, pl.ds(L, L)] * brows_v[row, pl.ds(L, L)])
                s = jnp.sum(t)
                acc = acc + jnp.where(lane == r, s, 0.0)
            res_v[pl.ds(g * L, L)] = 1.0 / (1.0 + jnp.exp(-acc))
            return carry

        lax.fori_loop(0, G, group, 0)
        pltpu.sync_copy(res_v, out_hbm.at[pl.ds(base, BW)])

    return k


def kernel(inputs, user_emb, user_bias, book_emb, book_bias):
    k = _build_kernel(user_emb.shape[0], book_emb.shape[0])
    uidx = inputs[:, 0]
    bidx = inputs[:, 1]
    out = k(uidx, bidx, user_emb, user_bias.reshape(-1),
            book_emb, book_bias.reshape(-1))
    return out.reshape(BATCH, 1)
